# Initial kernel scaffold; baseline (speedup 1.0000x reference)
import jax
import jax.numpy as jnp
from jax.experimental import pallas as pl


def kernel(site_energy, segment_ids, num_crystals):
    raise NotImplementedError("write your pallas kernel here")


# speedup vs baseline: 28.6712x; 28.6712x over previous
import functools

import jax
import jax.numpy as jnp
from jax import lax
from jax.experimental import pallas as pl
from jax.experimental.pallas import tpu as pltpu
from jax.experimental.pallas import tpu_sc as plsc

N_SITES = 6400000
N_CRYSTALS = 100000
NC, NS = 2, 16
NW = NC * NS
SITES_PER_W = N_SITES // NW
PAD_SEG = 100352
SEG_PER_TILE = PAD_SEG // NS
CHUNK = 8192
N_FULL_CHUNKS = SITES_PER_W // CHUNK
REM = SITES_PER_W - N_FULL_CHUNKS * CHUNK

_MESH = plsc.VectorSubcoreMesh(core_axis_name="c", subcore_axis_name="s")


@functools.partial(
    pl.kernel,
    out_type=jax.ShapeDtypeStruct((NC * PAD_SEG,), jnp.float32),
    mesh=_MESH,
    scratch_types=[
        pltpu.VMEM_SHARED((PAD_SEG,), jnp.float32),
        pltpu.VMEM((CHUNK,), jnp.int32),
        pltpu.VMEM((CHUNK,), jnp.float32),
        pltpu.VMEM((REM,), jnp.int32),
        pltpu.VMEM((REM,), jnp.float32),
        pltpu.VMEM((SEG_PER_TILE,), jnp.float32),
    ],
)
def _sc_partial(en_hbm, ids_hbm, zeros_hbm, out_hbm,
                acc, ids_v, en_v, ids_r, en_r, cbuf):
    c = lax.axis_index("c")
    s = lax.axis_index("s")
    wid = c * NS + s

    pltpu.sync_copy(zeros_hbm.at[pl.ds(s * SEG_PER_TILE, SEG_PER_TILE)],
                    acc.at[pl.ds(s * SEG_PER_TILE, SEG_PER_TILE)])
    plsc.subcore_barrier()

    base = wid * SITES_PER_W

    def chunk_body(i, _):
        start = base + i * CHUNK
        pltpu.sync_copy(ids_hbm.at[pl.ds(start, CHUNK)], ids_v)
        pltpu.sync_copy(en_hbm.at[pl.ds(start, CHUNK)], en_v)
        pltpu.sync_copy(en_v, acc.at[ids_v], add=True)
        return ()

    lax.fori_loop(0, N_FULL_CHUNKS, chunk_body, ())

    rstart = base + N_FULL_CHUNKS * CHUNK
    pltpu.sync_copy(ids_hbm.at[pl.ds(rstart, REM)], ids_r)
    pltpu.sync_copy(en_hbm.at[pl.ds(rstart, REM)], en_r)
    pltpu.sync_copy(en_r, acc.at[ids_r], add=True)

    plsc.subcore_barrier()

    off = s * SEG_PER_TILE
    pltpu.sync_copy(acc.at[pl.ds(off, SEG_PER_TILE)], cbuf)
    pltpu.sync_copy(cbuf, out_hbm.at[pl.ds(c * PAD_SEG + off, SEG_PER_TILE)])


SEG_PER_W = PAD_SEG // NW


@functools.partial(
    pl.kernel,
    out_type=jax.ShapeDtypeStruct((PAD_SEG,), jnp.float32),
    mesh=_MESH,
    scratch_types=[
        pltpu.VMEM((SEG_PER_W,), jnp.float32),
        pltpu.VMEM((SEG_PER_W,), jnp.float32),
    ],
)
def _sc_combine(partial_hbm, out_hbm, buf0, buf1):
    c = lax.axis_index("c")
    s = lax.axis_index("s")
    wid = c * NS + s
    off = wid * SEG_PER_W
    pltpu.sync_copy(partial_hbm.at[pl.ds(off, SEG_PER_W)], buf0)
    pltpu.sync_copy(partial_hbm.at[pl.ds(PAD_SEG + off, SEG_PER_W)], buf1)
    for j in range(SEG_PER_W // 16):
        sl = pl.ds(j * 16, 16)
        buf0[sl] = buf0[sl] + buf1[sl]
    pltpu.sync_copy(buf0, out_hbm.at[pl.ds(off, SEG_PER_W)])


def kernel(site_energy, segment_ids, num_crystals):
    en = site_energy.reshape(N_SITES)
    ids = segment_ids.reshape(N_SITES)
    zeros = jnp.zeros((PAD_SEG,), jnp.float32)
    partial = _sc_partial(en, ids, zeros)
    pooled = _sc_combine(partial)
    return pooled[:N_CRYSTALS, None]
